# Initial kernel scaffold; baseline (speedup 1.0000x reference)
import jax
import jax.numpy as jnp
from jax.experimental import pallas as pl


def kernel(context_ids, doc_ids, target_noise_ids, D, W, O):
    raise NotImplementedError("write your pallas kernel here")


# speedup vs baseline: 1.1962x; 1.1962x over previous
import functools

import jax
import jax.numpy as jnp
from jax import lax
from jax.experimental import pallas as pl
from jax.experimental.pallas import tpu as pltpu
from jax.experimental.pallas import tpu_sc as plsc

NC, NS, L = 2, 16, 16
NW = NC * NS
DIM = 64
CTX = 20
K = 6
NCH = DIM // L


def _transpose_tc(o):
    dim, v = o.shape
    blk = 1024

    def body(o_ref, ot_ref):
        ot_ref[...] = o_ref[...].T

    return pl.pallas_call(
        body,
        grid=(pl.cdiv(v, blk),),
        in_specs=[pl.BlockSpec((dim, blk), lambda i: (0, i))],
        out_specs=pl.BlockSpec((blk, dim), lambda i: (i, 0)),
        out_shape=jax.ShapeDtypeStruct((v, dim), o.dtype),
    )(o)


@functools.partial(jax.jit, static_argnames=("bw",))
def _sc_forward(ctx_rs, doc_rs, tn_rs, d_tab, w_tab, ot_tab, *, bw):
    n_ctx_rows = ctx_rs.shape[1]
    n_tn_rows = tn_rs.shape[1]
    sub = 32
    nsub = bw // sub
    rows_per_sub = sub * CTX // 128

    mesh = plsc.VectorSubcoreMesh(
        core_axis_name="c", subcore_axis_name="s",
        num_cores=NC, num_subcores=NS)

    @functools.partial(
        pl.kernel,
        out_type=jax.ShapeDtypeStruct((NW * bw * K,), jnp.float32),
        mesh=mesh,
        compiler_params=pltpu.CompilerParams(
            needs_layout_passes=False, use_tc_tiling_on_sc=False),
        scratch_types=[
            pltpu.VMEM((n_ctx_rows, 128), jnp.int32),
            pltpu.VMEM((bw,), jnp.int32),
            pltpu.VMEM((n_tn_rows, 128), jnp.int32),
            pltpu.VMEM((sub * CTX, DIM), jnp.float32),
            pltpu.VMEM((bw, DIM), jnp.float32),
            pltpu.VMEM((bw, DIM), jnp.float32),
            pltpu.VMEM((bw * K, DIM), jnp.float32),
            pltpu.VMEM((bw * K,), jnp.float32),
            pltpu.VMEM((L * L,), jnp.float32),
            pltpu.SemaphoreType.DMA,
            pltpu.SemaphoreType.DMA,
            pltpu.SemaphoreType.DMA,
        ],
    )
    def k(ctx_hbm, doc_hbm, tn_hbm, d_hbm, w_hbm, ot_hbm, out_hbm,
          ctx_i, doc_i, tn_i, w_rows, d_rows, x_v, ot_rows, out_v, tr_v,
          sem_w, sem_d, sem_ot):
        wid = lax.axis_index("s") * NC + lax.axis_index("c")

        pltpu.sync_copy(ctx_hbm.at[wid], ctx_i)
        pltpu.sync_copy(doc_hbm.at[wid], doc_i)
        pltpu.sync_copy(tn_hbm.at[wid], tn_i)

        d_cp = pltpu.async_copy(d_hbm.at[doc_i], d_rows, sem_d)
        ot_cps = [
            pltpu.async_copy(ot_hbm.at[tn_i.at[r]],
                             ot_rows.at[pl.ds(r * 128, 128)], sem_ot)
            for r in range(n_tn_rows)
        ]
        d_cp.wait()

        for s in range(nsub):
            w_cps = [
                pltpu.async_copy(
                    w_hbm.at[ctx_i.at[s * rows_per_sub + t]],
                    w_rows.at[pl.ds(t * 128, 128)], sem_w)
                for t in range(rows_per_sub)
            ]
            for cp in w_cps:
                cp.wait()

            def acc_body(bl, _, s=s):
                b = s * sub + bl
                accs = tuple(d_rows[b, pl.ds(c * L, L)] for c in range(NCH))

                def jbody(j, accs, bl=bl):
                    r = bl * CTX + j
                    return tuple(accs[c] + w_rows[r, pl.ds(c * L, L)]
                                 for c in range(NCH))

                accs = lax.fori_loop(0, CTX, jbody, accs)
                for c in range(NCH):
                    x_v[b, pl.ds(c * L, L)] = accs[c]
                return 0

            lax.fori_loop(0, sub, acc_body, 0)

        for cp in ot_cps:
            cp.wait()

        iota = lax.iota(jnp.int32, L)

        def dot_body(g, _):
            for i in range(L):
                p = g * L + i
                b = p // K
                pr = (x_v[b, pl.ds(0, L)] * ot_rows[p, pl.ds(0, L)])
                for c in range(1, NCH):
                    pr = pr + (x_v[b, pl.ds(c * L, L)]
                               * ot_rows[p, pl.ds(c * L, L)])
                plsc.store_scatter(tr_v, [iota * L + i], pr)
            s = tr_v[pl.ds(0, L)]
            for r in range(1, L):
                s = s + tr_v[pl.ds(r * L, L)]
            out_v[pl.ds(g * L, L)] = s
            return 0

        lax.fori_loop(0, bw * K // L, dot_body, 0)

        pltpu.sync_copy(out_v, out_hbm.at[pl.ds(wid * bw * K, bw * K)])

    return k(ctx_rs, doc_rs, tn_rs, d_tab, w_tab, ot_tab)


def kernel(context_ids, doc_ids, target_noise_ids, D, W, O):
    b = context_ids.shape[0]
    bw = b // NW
    ot = _transpose_tc(O)
    ctx_rs = context_ids.reshape(NW, bw * CTX // 128, 128)
    doc_rs = doc_ids.reshape(NW, bw)
    tn_rs = target_noise_ids.reshape(NW, bw * K // 128, 128)
    out = _sc_forward(ctx_rs, doc_rs, tn_rs, D, W, ot, bw=bw)
    return out.reshape(b, K)


# speedup vs baseline: 1.5163x; 1.2677x over previous
import functools

import jax
import jax.numpy as jnp
from jax import lax
from jax.experimental import pallas as pl
from jax.experimental.pallas import tpu as pltpu
from jax.experimental.pallas import tpu_sc as plsc

NC, NS, L = 2, 16, 16
NW = NC * NS
DIM = 64
CTX = 20
K = 6
NCH = DIM // L


@functools.partial(jax.jit, static_argnames=("bw",))
def _sc_forward(ctx_t, doc_ids, tn_t, d_tab, w_tab, ot_tab, *, bw):
    njc = 5
    npass = CTX // njc

    mesh = plsc.VectorSubcoreMesh(
        core_axis_name="c", subcore_axis_name="s",
        num_cores=NC, num_subcores=NS)

    @functools.partial(
        pl.kernel,
        out_type=jax.ShapeDtypeStruct((NW * bw * K,), jnp.float32),
        mesh=mesh,
        compiler_params=pltpu.CompilerParams(
            needs_layout_passes=False, use_tc_tiling_on_sc=False),
        scratch_types=[
            pltpu.VMEM((CTX, bw), jnp.int32),
            pltpu.VMEM((bw,), jnp.int32),
            pltpu.VMEM((K, bw), jnp.int32),
            pltpu.VMEM((njc, bw, DIM), jnp.float32),
            pltpu.VMEM((bw, DIM), jnp.float32),
            pltpu.VMEM((bw, DIM), jnp.float32),
            pltpu.VMEM((bw * K, DIM), jnp.float32),
            pltpu.VMEM((bw * K,), jnp.float32),
            pltpu.VMEM((L * L,), jnp.float32),
            pltpu.SemaphoreType.DMA,
            pltpu.SemaphoreType.DMA,
            pltpu.SemaphoreType.DMA,
        ],
    )
    def k(ctx_hbm, doc_hbm, tn_hbm, d_hbm, w_hbm, ot_hbm, out_hbm,
          ctx_i, doc_i, tn_i, w_rows, d_rows, x_v, ot_rows, out_v, tr_v,
          sem_w, sem_d, sem_ot):
        wid = lax.axis_index("s") * NC + lax.axis_index("c")
        base = wid * bw

        pltpu.sync_copy(ctx_hbm.at[:, pl.ds(base, bw)], ctx_i)
        pltpu.sync_copy(doc_hbm.at[pl.ds(base, bw)], doc_i)
        pltpu.sync_copy(tn_hbm.at[:, pl.ds(base, bw)], tn_i)

        d_cp = pltpu.async_copy(d_hbm.at[doc_i], d_rows, sem_d)
        ot_cps = [
            pltpu.async_copy(ot_hbm.at[tn_i.at[r]],
                             ot_rows.at[pl.ds(r * bw, bw)], sem_ot)
            for r in range(K)
        ]
        d_cp.wait()

        for p in range(npass):
            w_cps = [
                pltpu.async_copy(w_hbm.at[ctx_i.at[p * njc + t]],
                                 w_rows.at[t], sem_w)
                for t in range(njc)
            ]
            for cp in w_cps:
                cp.wait()

            def acc_body(b, _, p=p):
                src = d_rows if p == 0 else x_v
                for c in range(NCH):
                    acc = src[b, pl.ds(c * L, L)]
                    for t in range(njc):
                        acc = acc + w_rows[t, b, pl.ds(c * L, L)]
                    x_v[b, pl.ds(c * L, L)] = acc
                return 0

            lax.fori_loop(0, bw, acc_body, 0)

        for cp in ot_cps:
            cp.wait()

        iota = lax.iota(jnp.int32, L)

        def dot_body(g, _):
            for i in range(L):
                pr_idx = g * L + i
                b = pr_idx % bw
                pr = (x_v[b, pl.ds(0, L)] * ot_rows[pr_idx, pl.ds(0, L)])
                for c in range(1, NCH):
                    pr = pr + (x_v[b, pl.ds(c * L, L)]
                               * ot_rows[pr_idx, pl.ds(c * L, L)])
                plsc.store_scatter(tr_v, [iota * L + i], pr)
            s = tr_v[pl.ds(0, L)]
            for r in range(1, L):
                s = s + tr_v[pl.ds(r * L, L)]
            out_v[pl.ds(g * L, L)] = s
            return 0

        lax.fori_loop(0, bw * K // L, dot_body, 0)

        pltpu.sync_copy(out_v, out_hbm.at[pl.ds(wid * bw * K, bw * K)])

    return k(ctx_t, doc_ids, tn_t, d_tab, w_tab, ot_tab)


def kernel(context_ids, doc_ids, target_noise_ids, D, W, O):
    b = context_ids.shape[0]
    bw = b // NW
    out = _sc_forward(context_ids.T, doc_ids, target_noise_ids.T,
                      D, W, O.T, bw=bw)
    return out.reshape(NW, K, bw).transpose(0, 2, 1).reshape(b, K)
